# baseline (speedup 1.0000x reference)
import functools

import jax
import jax.numpy as jnp
from jax import lax
from jax.experimental import pallas as pl
from jax.experimental.pallas import tpu as pltpu
from jax.experimental.pallas import tpu_sc as plsc

NC, NS, LANES = 2, 16, 16
NW = NC * NS

B, L, DIM = 4096, 200, 64
N = B * L
QD = 4 * DIM
NQ = N // 4
NPW = NQ // NW
CH = 128
NCHUNK = NPW // CH
XCH = 1600
NXCH = NPW // XCH

_mesh = plsc.VectorSubcoreMesh(
    core_axis_name="c", subcore_axis_name="s", num_cores=NC, num_subcores=NS)


@functools.partial(
    pl.kernel,
    out_type=jax.ShapeDtypeStruct((NQ, QD), jnp.float32),
    mesh=_mesh,
    scratch_types=[
        [pltpu.VMEM((XCH,), jnp.int32) for _ in range(8)],
        pltpu.VMEM((NPW,), jnp.int32),
        pltpu.VMEM((CH, QD), jnp.float32),
        pltpu.VMEM((CH, QD), jnp.float32),
        pltpu.VMEM_SHARED((625, QD), jnp.float32),
        pltpu.SemaphoreType.DMA,
        pltpu.SemaphoreType.DMA,
        pltpu.SemaphoreType.DMA,
        pltpu.SemaphoreType.DMA,
    ],
    compiler_params=pltpu.CompilerParams(use_tc_tiling_on_sc=False),
)
def _sc_embed(b0, b1, b2, b3, r0, r1, r2, r3, qemb_hbm, out_hbm, brbuf,
              qidxb, rows0, rows1, emb_sh, gs0, gs1, ss0, ss1):
    planes = (b0, b1, b2, b3, r0, r1, r2, r3)
    wid = lax.axis_index("s") * NC + lax.axis_index("c")
    base = wid * NPW

    @pl.when(lax.axis_index("s") == 0)
    def _():
        pltpu.sync_copy(qemb_hbm, emb_sh)

    plsc.subcore_barrier()

    def xloop(xc, carry):
        for p in range(8):
            pltpu.sync_copy(
                planes[p].at[pl.ds(base + xc * XCH, XCH)], brbuf[p])

        def jloop(j, c2):
            q = jnp.zeros((LANES,), jnp.int32)
            for p in range(4):
                brick = brbuf[p][pl.ds(j * LANES, LANES)]
                rot = brbuf[4 + p][pl.ds(j * LANES, LANES)]
                idx = (1 + brick) * (1 + ((rot * 3) >> 8))
                q = q * 5 + idx
            qidxb[pl.ds(xc * XCH + j * LANES, LANES)] = q
            return c2

        return lax.fori_loop(0, XCH // LANES, jloop, carry)

    lax.fori_loop(0, NXCH, xloop, 0)

    def chunk_body(c, buf, g_sem, s_sem):
        row0 = base + c * CH
        @pl.when(c >= 2)
        def _():
            pltpu.make_async_copy(
                buf, out_hbm.at[pl.ds(row0, CH)], s_sem).wait()

        qidxs = qidxb.at[pl.ds(c * CH, CH)]
        pltpu.async_copy(emb_sh.at[qidxs], buf, g_sem).wait()
        pltpu.async_copy(buf, out_hbm.at[pl.ds(row0, CH)], s_sem)

    def pair(p, carry):
        chunk_body(2 * p, rows0, gs0, ss0)
        chunk_body(2 * p + 1, rows1, gs1, ss1)
        return carry

    lax.fori_loop(0, NCHUNK // 2, pair, 0)
    pltpu.make_async_copy(rows0, out_hbm.at[pl.ds(base, CH)], ss0).wait()
    pltpu.make_async_copy(rows1, out_hbm.at[pl.ds(base, CH)], ss1).wait()


def kernel(x, emb):
    xi = x.astype(jnp.int32)
    brick = xi[..., 0].reshape(NQ, 4)
    rot = xi[..., 1].reshape(NQ, 4)
    bp = [brick[:, p] for p in range(4)]
    rp = [rot[:, p] for p in range(4)]
    qemb = jnp.concatenate([
        jnp.broadcast_to(emb[:, None, None, None, :], (5, 5, 5, 5, DIM)),
        jnp.broadcast_to(emb[None, :, None, None, :], (5, 5, 5, 5, DIM)),
        jnp.broadcast_to(emb[None, None, :, None, :], (5, 5, 5, 5, DIM)),
        jnp.broadcast_to(emb[None, None, None, :, :], (5, 5, 5, 5, DIM)),
    ], axis=-1).reshape(625, QD)
    out = _sc_embed(*bp, *rp, qemb)
    return out.reshape(B, L, DIM)
